# Initial kernel scaffold; baseline (speedup 1.0000x reference)
import jax
import jax.numpy as jnp
from jax.experimental import pallas as pl


def kernel(att_map):
    raise NotImplementedError("write your pallas kernel here")


# speedup vs baseline: 45.0004x; 45.0004x over previous
import jax
import jax.numpy as jnp
from jax.experimental import pallas as pl
from jax.experimental.pallas import tpu as pltpu

_NB = 50
_H = 512
_W = 512


def _body(att_ref, out_ref):
    att = att_ref[0]
    idx = jnp.clip((att * _NB).astype(jnp.int32), 0, _NB - 1)
    counts = [jnp.sum((idx == b).astype(jnp.float32)) for b in range(_NB)]

    m = jnp.float32(-1.0)
    ind_max = jnp.int32(0)
    for b in range(_NB):
        better = counts[b] > m
        m = jnp.where(better, counts[b], m)
        ind_max = jnp.where(better, jnp.int32(b), ind_max)
    m2 = jnp.float32(-jnp.inf)
    ind_sec = jnp.int32(0)
    for b in range(_NB):
        v = jnp.where(jnp.int32(b) > ind_max, counts[b], jnp.float32(-1.0))
        better = v > m2
        m2 = jnp.where(better, v, m2)
        ind_sec = jnp.where(better, jnp.int32(b), ind_sec)

    thr = ind_sec.astype(jnp.float32) / _NB
    mask = att > thr
    area = jnp.sum(mask.astype(jnp.float32))
    value = jnp.maximum(jnp.sqrt(jnp.sqrt(area)), 1.0)
    inv = 1.0 / value
    supp = jnp.exp(jnp.log(jnp.clip(att, 1e-6, 1.0)) * inv)
    out_ref[0] = jnp.where(mask, supp, att)


def kernel(att_map):
    B = att_map.shape[0]
    x = att_map.reshape(B, _H, _W)
    out = pl.pallas_call(
        _body,
        grid=(B,),
        in_specs=[pl.BlockSpec((1, _H, _W), lambda i: (i, 0, 0))],
        out_specs=pl.BlockSpec((1, _H, _W), lambda i: (i, 0, 0)),
        out_shape=jax.ShapeDtypeStruct((B, _H, _W), jnp.float32),
        compiler_params=pltpu.CompilerParams(
            dimension_semantics=("arbitrary",)),
    )(x)
    return jax.lax.stop_gradient(out.reshape(att_map.shape))
